# Initial kernel scaffold; baseline (speedup 1.0000x reference)
import jax
import jax.numpy as jnp
from jax.experimental import pallas as pl


def kernel(x, edge_index, homophily_cond, batch, params):
    raise NotImplementedError("write your pallas kernel here")


# speedup vs baseline: 6.3218x; 6.3218x over previous
import functools

import jax
import jax.numpy as jnp
import numpy as np
from jax import lax
from jax.experimental import pallas as pl
from jax.experimental.pallas import tpu as pltpu
from jax.experimental.pallas import tpu_sc as plsc

N = 10000
FEAT = 256
LAT = 64
BNK = float(1.0 / np.sqrt(1.0 + 1e-5))

NT = 32
ROWS_PT = 320
N_OUT = NT * ROWS_PT
ACC_PT = 328
JUNKL = 320
EB = 64
JPAD = 128

E_MSG = 160000
BLK = 1024
NBLK = 158
E_PAD = NBLK * BLK
PAD_DST = 1 << 30
CAP = (NBLK + 2) * BLK
LBUF = 2080


def _sc_mesh():
    return plsc.VectorSubcoreMesh(core_axis_name="c", subcore_axis_name="s")


_SC_PARAMS = pltpu.CompilerParams(needs_layout_passes=False)


@functools.partial(
    pl.kernel,
    out_type=[
        jax.ShapeDtypeStruct((NT * CAP,), jnp.int32),
        jax.ShapeDtypeStruct((NT * CAP,), jnp.int32),
        jax.ShapeDtypeStruct((NT * 16,), jnp.int32),
        jax.ShapeDtypeStruct((N_OUT, 16), jnp.float32),
    ],
    mesh=_sc_mesh(),
    scratch_types=[
        pltpu.VMEM((BLK,), jnp.int32),
        pltpu.VMEM((BLK,), jnp.int32),
        pltpu.VMEM((LBUF,), jnp.int32),
        pltpu.VMEM((LBUF,), jnp.int32),
        pltpu.VMEM((ACC_PT, 16), jnp.float32),
        pltpu.VMEM((16,), jnp.int32),
    ],
    compiler_params=_SC_PARAMS,
)
def _part_call(src_hbm, dst_hbm, srcs_out, dstl_out, cnt_out, deg_out,
               src_st, dst_st, sbuf, dbuf, degf, cnt_v):
    w = lax.axis_index("c") * 16 + lax.axis_index("s")
    base = w * ROWS_PT
    zrow = jnp.zeros((16,), jnp.float32)

    def zdeg(i, carry):
        degf[i, :] = zrow
        return carry

    lax.fori_loop(0, ACC_PT, zdeg, 0)

    ones_f = jnp.ones((16,), jnp.float32)
    zcol = jnp.zeros((16,), jnp.int32)

    base_v = jnp.broadcast_to(base, (16,)).astype(jnp.int32)

    def blk(b, carry):
        cnt, gp = carry
        off = b * BLK
        pltpu.sync_copy(src_hbm.at[pl.ds(off, BLK)], src_st)
        pltpu.sync_copy(dst_hbm.at[pl.ds(off, BLK)], dst_st)
        for k in range(BLK // 16):
            d = dst_st[pl.ds(k * 16, 16)]
            sv = src_st[pl.ds(k * 16, 16)]
            l = d - base_v
            m = (l >= 0) & (l < ROWS_PT)
            plsc.store_compressed(sbuf.at[pl.ds(cnt, 16)], sv, mask=m)
            plsc.store_compressed(dbuf.at[pl.ds(cnt, 16)], l, mask=m)
            plsc.addupdate_scatter(degf, [l, zcol], ones_f, mask=m)
            pc = plsc.all_reduce_population_count(m)
            cnt = cnt + (pc[0] if pc.ndim else pc)
        spi = lax.div(cnt, BLK)

        @pl.when(spi > 0)
        def _():
            pltpu.sync_copy(sbuf.at[pl.ds(0, BLK)],
                            srcs_out.at[pl.ds(w * CAP + gp * BLK, BLK)])
            pltpu.sync_copy(dbuf.at[pl.ds(0, BLK)],
                            dstl_out.at[pl.ds(w * CAP + gp * BLK, BLK)])
            for k in range((LBUF - BLK) // 16):
                sbuf[pl.ds(k * 16, 16)] = sbuf[pl.ds(BLK + k * 16, 16)]
                dbuf[pl.ds(k * 16, 16)] = dbuf[pl.ds(BLK + k * 16, 16)]

        return (cnt - spi * BLK, gp + spi)

    cnt, gp = lax.fori_loop(0, NBLK, blk, (jnp.int32(0), jnp.int32(0)))
    total = gp * BLK + cnt
    for k in range(JPAD // 16):
        sbuf[pl.ds(cnt + k * 16, 16)] = zcol
        dbuf[pl.ds(cnt + k * 16, 16)] = jnp.full((16,), JUNKL, jnp.int32)
    pltpu.sync_copy(sbuf.at[pl.ds(0, BLK)],
                    srcs_out.at[pl.ds(w * CAP + gp * BLK, BLK)])
    pltpu.sync_copy(dbuf.at[pl.ds(0, BLK)],
                    dstl_out.at[pl.ds(w * CAP + gp * BLK, BLK)])
    pltpu.sync_copy(sbuf.at[pl.ds(BLK, BLK)],
                    srcs_out.at[pl.ds(w * CAP + (gp + 1) * BLK, BLK)])
    pltpu.sync_copy(dbuf.at[pl.ds(BLK, BLK)],
                    dstl_out.at[pl.ds(w * CAP + (gp + 1) * BLK, BLK)])
    iota = lax.iota(jnp.int32, 16)
    cnt_v[:] = jnp.where(iota == 0, jnp.broadcast_to(total, (16,)),
                         jnp.zeros((16,), jnp.int32))
    pltpu.sync_copy(cnt_v, cnt_out.at[pl.ds(w * 16, 16)])
    pltpu.sync_copy(degf.at[pl.ds(0, ROWS_PT)],
                    deg_out.at[pl.ds(base, ROWS_PT)])


LCH = 1024


@functools.partial(
    pl.kernel,
    out_type=jax.ShapeDtypeStruct((N_OUT, FEAT), jnp.float32),
    mesh=_sc_mesh(),
    scratch_types=[
        pltpu.VMEM((ACC_PT, FEAT), jnp.float32),
        pltpu.VMEM((LCH,), jnp.int32),
        pltpu.VMEM((LCH,), jnp.int32),
        pltpu.VMEM((EB, FEAT), jnp.float32),
        pltpu.VMEM((EB, FEAT), jnp.float32),
        pltpu.VMEM((16,), jnp.int32),
        pltpu.SemaphoreType.DMA,
        pltpu.SemaphoreType.DMA,
    ],
    compiler_params=_SC_PARAMS,
)
def _scatter_call(hw_hbm, srcs_hbm, dstl_hbm, cnt_hbm, zacc_hbm, out_hbm,
                  acc, sl, dl, rows_a, rows_b, cnt_v, sem_a, sem_b):
    w = lax.axis_index("c") * 16 + lax.axis_index("s")
    base = w * ROWS_PT
    pltpu.sync_copy(zacc_hbm, acc)
    pltpu.sync_copy(cnt_hbm.at[pl.ds(w * 16, 16)], cnt_v)
    total = cnt_v[:][0]
    nlc = jnp.maximum(lax.div(total + (LCH - 1), LCH), 1)

    def gath(b, rv, sm):
        pltpu.async_copy(hw_hbm.at[sl.at[pl.ds(b * EB, EB)]], rv, sm)

    def drain(rv, sm):
        pltpu.make_async_copy(hw_hbm.at[pl.ds(0, EB)], rv, sm).wait()

    def accum(boff, rv_ref):
        def grp(b, c2):
            dvec = dl[pl.ds(boff + b * 16, 16)]
            for j in range(16):
                d = dvec[j]
                e = b * 16 + j

                @plsc.parallel_loop(0, FEAT, 16, unroll=16)
                def _(q):
                    plsc.addupdate(acc.at[d, pl.ds(q, 16)],
                                   rv_ref[e, pl.ds(q, 16)])
            return c2

        lax.fori_loop(0, EB // 16, grp, 0)

    def chunk(ch, carry):
        off = w * CAP + ch * LCH
        pltpu.sync_copy(srcs_hbm.at[pl.ds(off, LCH)], sl)
        pltpu.sync_copy(dstl_hbm.at[pl.ds(off, LCH)], dl)
        rem = total - ch * LCH
        nbc = jnp.minimum(lax.div(rem + (EB - 1), EB), LCH // EB)
        npr = lax.div(nbc + 1, 2)
        gath(0, rows_a, sem_a)
        gath(1, rows_b, sem_b)

        def pair(g, c2):
            drain(rows_a, sem_a)
            accum(2 * g * EB, rows_a)

            @pl.when(g + 1 < npr)
            def _():
                gath(2 * g + 2, rows_a, sem_a)

            drain(rows_b, sem_b)
            accum((2 * g + 1) * EB, rows_b)

            @pl.when(g + 1 < npr)
            def _():
                gath(2 * g + 3, rows_b, sem_b)

            return c2

        lax.fori_loop(0, npr, pair, 0)
        return carry

    lax.fori_loop(0, nlc, chunk, 0)
    pltpu.sync_copy(acc.at[pl.ds(0, ROWS_PT)],
                    out_hbm.at[pl.ds(base, ROWS_PT)])


def _dis(deg_ref):
    return lax.rsqrt(deg_ref[:, 0:1] + 1.0)


def _mmscale_body(x_ref, w_ref, deg_ref, o_ref):
    o_ref[...] = jnp.dot(x_ref[...], w_ref[...],
                         preferred_element_type=jnp.float32) * _dis(deg_ref)


def _mmscale(x, w, deg16):
    T = 1000
    return pl.pallas_call(
        _mmscale_body,
        grid=(N // T,),
        in_specs=[
            pl.BlockSpec((T, FEAT), lambda i: (i, 0)),
            pl.BlockSpec((FEAT, FEAT), lambda i: (0, 0)),
            pl.BlockSpec((T, 16), lambda i: (i, 0)),
        ],
        out_specs=pl.BlockSpec((T, FEAT), lambda i: (i, 0)),
        out_shape=jax.ShapeDtypeStruct((N, FEAT), jnp.float32),
    )(x, w, deg16)


def _hcvec(hom_ref, w3_ref, b_ref):
    return (hom_ref[0, 0] * w3_ref[0:1, :] + hom_ref[0, 1] * w3_ref[1:2, :]
            + hom_ref[0, 2] * w3_ref[2:3, :] + b_ref[...])


def _gcn_post(hom_ref, acc_ref, hws_ref, dis, gb_ref, g_ref, bb_ref,
              hw3_ref, hb_ref):
    pre = dis * (acc_ref[...] + hws_ref[...]) + gb_ref[...]
    h = jnp.maximum(pre * (g_ref[...] * BNK) + bb_ref[...], 0.0)
    return h + _hcvec(hom_ref, hw3_ref, hb_ref)


def _layer_body(hom_ref, acc_ref, hws_ref, deg_ref, gb_ref, g_ref, bb_ref,
                hw3_ref, hb_ref, w_ref, o_ref):
    dis = _dis(deg_ref)
    h = _gcn_post(hom_ref, acc_ref, hws_ref, dis, gb_ref, g_ref, bb_ref,
                  hw3_ref, hb_ref)
    o_ref[...] = jnp.dot(h, w_ref[...],
                         preferred_element_type=jnp.float32) * dis


def _layer(hom, acc, hws, deg16, gb, g, bb, hw3, hb, w):
    T = 1000
    full = lambda r, c: pl.BlockSpec((r, c), lambda i: (0, 0))
    return pl.pallas_call(
        _layer_body,
        grid=(N // T,),
        in_specs=[
            pl.BlockSpec(memory_space=pltpu.SMEM),
            pl.BlockSpec((T, FEAT), lambda i: (i, 0)),
            pl.BlockSpec((T, FEAT), lambda i: (i, 0)),
            pl.BlockSpec((T, 16), lambda i: (i, 0)),
            full(1, FEAT), full(1, FEAT), full(1, FEAT),
            full(3, FEAT), full(1, FEAT), full(FEAT, FEAT),
        ],
        out_specs=pl.BlockSpec((T, FEAT), lambda i: (i, 0)),
        out_shape=jax.ShapeDtypeStruct((N, FEAT), jnp.float32),
    )(hom, acc, hws, deg16, gb, g, bb, hw3, hb, w)


def _final_body(hom_ref, acc_ref, hws_ref, deg_ref, gb_ref, g_ref, bb_ref,
                hw3_ref, hb_ref,
                muW_ref, muH_ref, mub_ref, lvW_ref, lvH_ref, lvb_ref,
                l1W_ref, l1b_ref, l2W_ref, l2b_ref,
                pW_ref, pb_ref, pg_ref, pbb_ref,
                t1W_ref, t1b_ref, t2W_ref, t2b_ref,
                z_ref, lv_ref, y_ref, xr_ref, zs_ref):
    i = pl.program_id(0)
    dis = _dis(deg_ref)
    h = _gcn_post(hom_ref, acc_ref, hws_ref, dis, gb_ref, g_ref, bb_ref,
                  hw3_ref, hb_ref)
    mub = _hcvec(hom_ref, muH_ref, mub_ref)
    lvb = _hcvec(hom_ref, lvH_ref, lvb_ref)
    z = jnp.dot(h, muW_ref[...], preferred_element_type=jnp.float32) + mub
    z_ref[...] = z
    lv_ref[...] = jnp.dot(h, lvW_ref[...],
                          preferred_element_type=jnp.float32) + lvb
    t = jnp.maximum(jnp.dot(z, l1W_ref[...],
                            preferred_element_type=jnp.float32)
                    + l1b_ref[...], 0.0)
    y_ref[...] = jnp.dot(t, l2W_ref[...],
                         preferred_element_type=jnp.float32) + l2b_ref[...]
    zp = (jnp.dot(z, pW_ref[...], preferred_element_type=jnp.float32)
          + pb_ref[...]) * (pg_ref[...] * BNK) + pbb_ref[...]
    xr = jnp.maximum(jnp.dot(zp, t1W_ref[...],
                             preferred_element_type=jnp.float32)
                     + t1b_ref[...], 0.0)
    xr_ref[...] = jnp.dot(xr, t2W_ref[...],
                          preferred_element_type=jnp.float32) + t2b_ref[...]
    part = jnp.sum(z, axis=0, keepdims=True)

    @pl.when(i == 0)
    def _():
        zs_ref[...] = part

    @pl.when(i != 0)
    def _():
        zs_ref[...] = zs_ref[...] + part


def _final(hom, acc, hws, deg16, gb, g, bb, hw3, hb,
           muW, muH, mub, lvW, lvH, lvb, l1W, l1b, l2W, l2b,
           pW, pb, pg, pbb, t1W, t1b, t2W, t2b):
    T = 1000
    NCLS = 7
    TLAT = 128
    full = lambda r, c: pl.BlockSpec((r, c), lambda i: (0, 0))
    row = lambda c: pl.BlockSpec((T, c), lambda i: (i, 0))
    return pl.pallas_call(
        _final_body,
        grid=(N // T,),
        in_specs=[
            pl.BlockSpec(memory_space=pltpu.SMEM),
            row(FEAT), row(FEAT), pl.BlockSpec((T, 16), lambda i: (i, 0)),
            full(1, FEAT), full(1, FEAT), full(1, FEAT),
            full(3, FEAT), full(1, FEAT),
            full(FEAT, LAT), full(3, LAT), full(1, LAT),
            full(FEAT, LAT), full(3, LAT), full(1, LAT),
            full(LAT, LAT), full(1, LAT), full(LAT, NCLS), full(1, NCLS),
            full(LAT, TLAT), full(1, TLAT), full(1, TLAT), full(1, TLAT),
            full(TLAT, FEAT), full(1, FEAT), full(FEAT, FEAT), full(1, FEAT),
        ],
        out_specs=[
            row(LAT), row(LAT), row(NCLS), row(FEAT),
            pl.BlockSpec((1, LAT), lambda i: (0, 0)),
        ],
        out_shape=[
            jax.ShapeDtypeStruct((N, LAT), jnp.float32),
            jax.ShapeDtypeStruct((N, LAT), jnp.float32),
            jax.ShapeDtypeStruct((N, NCLS), jnp.float32),
            jax.ShapeDtypeStruct((N, FEAT), jnp.float32),
            jax.ShapeDtypeStruct((1, LAT), jnp.float32),
        ],
    )(hom, acc, hws, deg16, gb, g, bb, hw3, hb,
      muW, muH, mub, lvW, lvH, lvb, l1W, l1b, l2W, l2b,
      pW, pb, pg, pbb, t1W, t1b, t2W, t2b)


def _adj_body(zr_ref, zc_ref, o_ref):
    o_ref[...] = jax.nn.sigmoid(
        lax.dot_general(zr_ref[...], zc_ref[...], (((1,), (1,)), ((), ())),
                        preferred_element_type=jnp.float32))


def _adj(z):
    TR, TCOL = 1000, 2048
    return pl.pallas_call(
        _adj_body,
        grid=(N // TR, pl.cdiv(N, TCOL)),
        in_specs=[
            pl.BlockSpec((TR, LAT), lambda i, j: (i, 0)),
            pl.BlockSpec((TCOL, LAT), lambda i, j: (j, 0)),
        ],
        out_specs=pl.BlockSpec((TR, TCOL), lambda i, j: (i, j)),
        out_shape=jax.ShapeDtypeStruct((N, N), jnp.float32),
    )(z, z)


def _heads_body(zs_ref, lh1, lh1b, lh2, lh2b, sh1, sh1b, sh2, sh2b,
                fh1, fh1b, fh2, fh2b, o_ref):
    zg = zs_ref[...] * (1.0 / N)

    def head(w1, b1, w2, b2):
        t = jnp.maximum(jnp.dot(zg, w1[...],
                                preferred_element_type=jnp.float32)
                        + b1[...], 0.0)
        return jnp.dot(t, w2[...], preferred_element_type=jnp.float32) + b2[...]

    lh = jax.nn.sigmoid(head(lh1, lh1b, lh2, lh2b))
    sh = jax.nn.sigmoid(head(sh1, sh1b, sh2, sh2b))
    fh = jnp.tanh(head(fh1, fh1b, fh2, fh2b))
    o_ref[...] = jnp.concatenate([lh, sh, fh], axis=1)


def _heads(zsum, lh1, lh1b, lh2, lh2b, sh1, sh1b, sh2, sh2b,
           fh1, fh1b, fh2, fh2b):
    full = lambda r, c: pl.BlockSpec((r, c), lambda: (0, 0))
    return pl.pallas_call(
        _heads_body,
        in_specs=[full(1, LAT),
                  full(LAT, LAT), full(1, LAT), full(LAT, 1), full(1, 1),
                  full(LAT, LAT), full(1, LAT), full(LAT, 1), full(1, 1),
                  full(LAT, LAT), full(1, LAT), full(LAT, 1), full(1, 1)],
        out_specs=full(1, 3),
        out_shape=jax.ShapeDtypeStruct((1, 3), jnp.float32),
    )(zsum, lh1, lh1b, lh2, lh2b, sh1, sh1b, sh2, sh2b, fh1, fh1b, fh2, fh2b)


def kernel(x, edge_index, homophily_cond, batch, params):
    p = params
    f32 = jnp.float32
    src = edge_index[0]
    dst = edge_index[1]
    src_p = jnp.concatenate([src, jnp.zeros((E_PAD - E_MSG,), jnp.int32)])
    dst_p = jnp.concatenate([dst, jnp.full((E_PAD - E_MSG,), PAD_DST,
                                           jnp.int32)])
    zacc = jnp.zeros((ACC_PT, FEAT), f32)
    hom = homophily_cond
    r = lambda v: v.reshape(1, -1)

    srcs, dstl, cnts, deg16 = _part_call(src_p, dst_p)
    hw0s = _mmscale(x, p['gcn0_W'], deg16)
    acc0 = _scatter_call(hw0s, srcs, dstl, cnts, zacc)
    hw1s = _layer(hom, acc0, hw0s, deg16,
                  r(p['gcn0_b']), r(p['bn0_g']), r(p['bn0_b']),
                  p['hom0_W'], r(p['hom0_b']), p['gcn1_W'])
    acc1 = _scatter_call(hw1s, srcs, dstl, cnts, zacc)
    z, lv, y, xr, zsum = _final(
        hom, acc1, hw1s, deg16,
        r(p['gcn1_b']), r(p['bn1_g']), r(p['bn1_b']),
        p['hom1_W'], r(p['hom1_b']),
        p['mu_W'][:FEAT], p['mu_W'][FEAT:], r(p['mu_b']),
        p['lv_W'][:FEAT], p['lv_W'][FEAT:], r(p['lv_b']),
        p['lab1_W'], r(p['lab1_b']), p['lab2_W'], r(p['lab2_b']),
        p['proj_W'], r(p['proj_b']), r(p['projbn_g']), r(p['projbn_b']),
        p['t1_W'], r(p['t1_b']), p['t2_W'], r(p['t2_b']))
    adj = _adj(z)
    homp = _heads(zsum,
                  p['lh1_W'], r(p['lh1_b']), p['lh2_W'], r(p['lh2_b']),
                  p['sh1_W'], r(p['sh1_b']), p['sh2_W'], r(p['sh2_b']),
                  p['fh1_W'], r(p['fh1_b']), p['fh2_W'], r(p['fh2_b']))
    return (adj, xr, y, homp, z, lv)


# speedup vs baseline: 7.0989x; 1.1229x over previous
import functools

import jax
import jax.numpy as jnp
import numpy as np
from jax import lax
from jax.experimental import pallas as pl
from jax.experimental.pallas import tpu as pltpu
from jax.experimental.pallas import tpu_sc as plsc

N = 10000
FEAT = 256
LAT = 64
BNK = float(1.0 / np.sqrt(1.0 + 1e-5))

NT = 32
ROWS_PT = 320
N_OUT = NT * ROWS_PT
ACC_PT = 328
JUNKL = 320
EB = 64
JPAD = 128

E_MSG = 160000
BLK = 1024
NBLK = 158
E_PAD = NBLK * BLK
PAD_DST = 1 << 30
CAP = (NBLK + 2) * BLK
LBUF = 2080


def _sc_mesh():
    return plsc.VectorSubcoreMesh(core_axis_name="c", subcore_axis_name="s")


_SC_PARAMS = pltpu.CompilerParams(needs_layout_passes=False)


@functools.partial(
    pl.kernel,
    out_type=[
        jax.ShapeDtypeStruct((NT * CAP,), jnp.int32),
        jax.ShapeDtypeStruct((NT * CAP,), jnp.int32),
        jax.ShapeDtypeStruct((NT * 16,), jnp.int32),
        jax.ShapeDtypeStruct((N_OUT, 16), jnp.float32),
    ],
    mesh=_sc_mesh(),
    scratch_types=[
        pltpu.VMEM((BLK,), jnp.int32),
        pltpu.VMEM((BLK,), jnp.int32),
        pltpu.VMEM((BLK,), jnp.int32),
        pltpu.VMEM((BLK,), jnp.int32),
        pltpu.VMEM((LBUF,), jnp.int32),
        pltpu.VMEM((LBUF,), jnp.int32),
        pltpu.VMEM((ACC_PT, 16), jnp.float32),
        pltpu.VMEM((16,), jnp.int32),
        pltpu.SemaphoreType.DMA,
        pltpu.SemaphoreType.DMA,
    ],
    compiler_params=_SC_PARAMS,
)
def _part_call(src_hbm, dst_hbm, srcs_out, dstl_out, cnt_out, deg_out,
               src_a, dst_a, src_b, dst_b, sbuf, dbuf, degf, cnt_v,
               sem_a, sem_b):
    w = lax.axis_index("c") * 16 + lax.axis_index("s")
    base = w * ROWS_PT
    zrow = jnp.zeros((16,), jnp.float32)

    def zdeg(i, carry):
        degf[i, :] = zrow
        return carry

    lax.fori_loop(0, ACC_PT, zdeg, 0)

    ones_f = jnp.ones((16,), jnp.float32)
    zcol = jnp.zeros((16,), jnp.int32)

    base_v = jnp.broadcast_to(base, (16,)).astype(jnp.int32)

    def fetch_blk(b, sv, dv, sm):
        pltpu.async_copy(src_hbm.at[pl.ds(b * BLK, BLK)], sv, sm)
        pltpu.async_copy(dst_hbm.at[pl.ds(b * BLK, BLK)], dv, sm)

    def drain_blk(sv, dv, sm):
        pltpu.make_async_copy(src_hbm.at[pl.ds(0, BLK)], sv, sm).wait()
        pltpu.make_async_copy(src_hbm.at[pl.ds(0, BLK)], dv, sm).wait()

    def scan(sv_ref, dv_ref, cnt, gp):
        for k in range(BLK // 16):
            d = dv_ref[pl.ds(k * 16, 16)]
            sv = sv_ref[pl.ds(k * 16, 16)]
            l = d - base_v
            m = (l >= 0) & (l < ROWS_PT)
            plsc.store_compressed(sbuf.at[pl.ds(cnt, 16)], sv, mask=m)
            plsc.store_compressed(dbuf.at[pl.ds(cnt, 16)], l, mask=m)
            plsc.addupdate_scatter(degf, [l, zcol], ones_f, mask=m)
            pc = plsc.all_reduce_population_count(m)
            cnt = cnt + (pc[0] if pc.ndim else pc)
        spi = lax.div(cnt, BLK)

        @pl.when(spi > 0)
        def _():
            pltpu.sync_copy(sbuf.at[pl.ds(0, BLK)],
                            srcs_out.at[pl.ds(w * CAP + gp * BLK, BLK)])
            pltpu.sync_copy(dbuf.at[pl.ds(0, BLK)],
                            dstl_out.at[pl.ds(w * CAP + gp * BLK, BLK)])
            for k in range((LBUF - BLK) // 16):
                sbuf[pl.ds(k * 16, 16)] = sbuf[pl.ds(BLK + k * 16, 16)]
                dbuf[pl.ds(k * 16, 16)] = dbuf[pl.ds(BLK + k * 16, 16)]

        return (cnt - spi * BLK, gp + spi)

    fetch_blk(0, src_a, dst_a, sem_a)
    fetch_blk(1, src_b, dst_b, sem_b)

    def pairblk(i, carry):
        cnt, gp = carry
        drain_blk(src_a, dst_a, sem_a)
        cnt, gp = scan(src_a, dst_a, cnt, gp)

        @pl.when(i + 1 < NBLK // 2)
        def _():
            fetch_blk(2 * i + 2, src_a, dst_a, sem_a)

        drain_blk(src_b, dst_b, sem_b)
        cnt, gp = scan(src_b, dst_b, cnt, gp)

        @pl.when(i + 1 < NBLK // 2)
        def _():
            fetch_blk(2 * i + 3, src_b, dst_b, sem_b)

        return (cnt, gp)

    cnt, gp = lax.fori_loop(0, NBLK // 2, pairblk,
                            (jnp.int32(0), jnp.int32(0)))
    total = gp * BLK + cnt
    for k in range(JPAD // 16):
        sbuf[pl.ds(cnt + k * 16, 16)] = zcol
        dbuf[pl.ds(cnt + k * 16, 16)] = jnp.full((16,), JUNKL, jnp.int32)
    pltpu.sync_copy(sbuf.at[pl.ds(0, BLK)],
                    srcs_out.at[pl.ds(w * CAP + gp * BLK, BLK)])
    pltpu.sync_copy(dbuf.at[pl.ds(0, BLK)],
                    dstl_out.at[pl.ds(w * CAP + gp * BLK, BLK)])
    pltpu.sync_copy(sbuf.at[pl.ds(BLK, BLK)],
                    srcs_out.at[pl.ds(w * CAP + (gp + 1) * BLK, BLK)])
    pltpu.sync_copy(dbuf.at[pl.ds(BLK, BLK)],
                    dstl_out.at[pl.ds(w * CAP + (gp + 1) * BLK, BLK)])
    iota = lax.iota(jnp.int32, 16)
    cnt_v[:] = jnp.where(iota == 0, jnp.broadcast_to(total, (16,)),
                         jnp.zeros((16,), jnp.int32))
    pltpu.sync_copy(cnt_v, cnt_out.at[pl.ds(w * 16, 16)])
    pltpu.sync_copy(degf.at[pl.ds(0, ROWS_PT)],
                    deg_out.at[pl.ds(base, ROWS_PT)])


LCH = 1024


@functools.partial(
    pl.kernel,
    out_type=jax.ShapeDtypeStruct((N_OUT, FEAT), jnp.float32),
    mesh=_sc_mesh(),
    scratch_types=[
        pltpu.VMEM((ACC_PT, FEAT), jnp.float32),
        pltpu.VMEM((LCH,), jnp.int32),
        pltpu.VMEM((LCH,), jnp.int32),
        pltpu.VMEM((EB, FEAT), jnp.float32),
        pltpu.VMEM((EB, FEAT), jnp.float32),
        pltpu.VMEM((16,), jnp.int32),
        pltpu.SemaphoreType.DMA,
        pltpu.SemaphoreType.DMA,
    ],
    compiler_params=_SC_PARAMS,
)
def _scatter_call(hw_hbm, srcs_hbm, dstl_hbm, cnt_hbm, zacc_hbm, out_hbm,
                  acc, sl, dl, rows_a, rows_b, cnt_v, sem_a, sem_b):
    w = lax.axis_index("c") * 16 + lax.axis_index("s")
    base = w * ROWS_PT
    pltpu.sync_copy(zacc_hbm, acc)
    pltpu.sync_copy(cnt_hbm.at[pl.ds(w * 16, 16)], cnt_v)
    total = cnt_v[:][0]
    nlc = jnp.maximum(lax.div(total + (LCH - 1), LCH), 1)

    def gath(b, rv, sm):
        pltpu.async_copy(hw_hbm.at[sl.at[pl.ds(b * EB, EB)]], rv, sm)

    def drain(rv, sm):
        pltpu.make_async_copy(hw_hbm.at[pl.ds(0, EB)], rv, sm).wait()

    def accum(boff, rv_ref):
        def grp(b, c2):
            dvec = dl[pl.ds(boff + b * 16, 16)]
            for j in range(16):
                d = dvec[j]
                e = b * 16 + j

                @plsc.parallel_loop(0, FEAT, 16, unroll=16)
                def _(q):
                    plsc.addupdate(acc.at[d, pl.ds(q, 16)],
                                   rv_ref[e, pl.ds(q, 16)])
            return c2

        lax.fori_loop(0, EB // 16, grp, 0)

    def chunk(ch, carry):
        off = w * CAP + ch * LCH
        pltpu.sync_copy(srcs_hbm.at[pl.ds(off, LCH)], sl)
        pltpu.sync_copy(dstl_hbm.at[pl.ds(off, LCH)], dl)
        rem = total - ch * LCH
        nbc = jnp.minimum(lax.div(rem + (EB - 1), EB), LCH // EB)
        npr = lax.div(nbc + 1, 2)
        gath(0, rows_a, sem_a)
        gath(1, rows_b, sem_b)

        def pair(g, c2):
            drain(rows_a, sem_a)
            accum(2 * g * EB, rows_a)

            @pl.when(g + 1 < npr)
            def _():
                gath(2 * g + 2, rows_a, sem_a)

            drain(rows_b, sem_b)
            accum((2 * g + 1) * EB, rows_b)

            @pl.when(g + 1 < npr)
            def _():
                gath(2 * g + 3, rows_b, sem_b)

            return c2

        lax.fori_loop(0, npr, pair, 0)
        return carry

    lax.fori_loop(0, nlc, chunk, 0)
    pltpu.sync_copy(acc.at[pl.ds(0, ROWS_PT)],
                    out_hbm.at[pl.ds(base, ROWS_PT)])


def _dis(deg_ref):
    return lax.rsqrt(deg_ref[:, 0:1] + 1.0)


def _mmscale_body(x_ref, w_ref, deg_ref, o_ref):
    o_ref[...] = jnp.dot(x_ref[...], w_ref[...],
                         preferred_element_type=jnp.float32) * _dis(deg_ref)


def _mmscale(x, w, deg16):
    T = 1000
    return pl.pallas_call(
        _mmscale_body,
        grid=(N // T,),
        in_specs=[
            pl.BlockSpec((T, FEAT), lambda i: (i, 0)),
            pl.BlockSpec((FEAT, FEAT), lambda i: (0, 0)),
            pl.BlockSpec((T, 16), lambda i: (i, 0)),
        ],
        out_specs=pl.BlockSpec((T, FEAT), lambda i: (i, 0)),
        out_shape=jax.ShapeDtypeStruct((N, FEAT), jnp.float32),
    )(x, w, deg16)


def _hcvec(hom_ref, w3_ref, b_ref):
    return (hom_ref[0, 0] * w3_ref[0:1, :] + hom_ref[0, 1] * w3_ref[1:2, :]
            + hom_ref[0, 2] * w3_ref[2:3, :] + b_ref[...])


def _gcn_post(hom_ref, acc_ref, hws_ref, dis, gb_ref, g_ref, bb_ref,
              hw3_ref, hb_ref):
    pre = dis * (acc_ref[...] + hws_ref[...]) + gb_ref[...]
    h = jnp.maximum(pre * (g_ref[...] * BNK) + bb_ref[...], 0.0)
    return h + _hcvec(hom_ref, hw3_ref, hb_ref)


def _layer_body(hom_ref, acc_ref, hws_ref, deg_ref, gb_ref, g_ref, bb_ref,
                hw3_ref, hb_ref, w_ref, o_ref):
    dis = _dis(deg_ref)
    h = _gcn_post(hom_ref, acc_ref, hws_ref, dis, gb_ref, g_ref, bb_ref,
                  hw3_ref, hb_ref)
    o_ref[...] = jnp.dot(h, w_ref[...],
                         preferred_element_type=jnp.float32) * dis


def _layer(hom, acc, hws, deg16, gb, g, bb, hw3, hb, w):
    T = 1000
    full = lambda r, c: pl.BlockSpec((r, c), lambda i: (0, 0))
    return pl.pallas_call(
        _layer_body,
        grid=(N // T,),
        in_specs=[
            pl.BlockSpec(memory_space=pltpu.SMEM),
            pl.BlockSpec((T, FEAT), lambda i: (i, 0)),
            pl.BlockSpec((T, FEAT), lambda i: (i, 0)),
            pl.BlockSpec((T, 16), lambda i: (i, 0)),
            full(1, FEAT), full(1, FEAT), full(1, FEAT),
            full(3, FEAT), full(1, FEAT), full(FEAT, FEAT),
        ],
        out_specs=pl.BlockSpec((T, FEAT), lambda i: (i, 0)),
        out_shape=jax.ShapeDtypeStruct((N, FEAT), jnp.float32),
    )(hom, acc, hws, deg16, gb, g, bb, hw3, hb, w)


def _final_body(hom_ref, acc_ref, hws_ref, deg_ref, gb_ref, g_ref, bb_ref,
                hw3_ref, hb_ref,
                muW_ref, muH_ref, mub_ref, lvW_ref, lvH_ref, lvb_ref,
                l1W_ref, l1b_ref, l2W_ref, l2b_ref,
                pW_ref, pb_ref, pg_ref, pbb_ref,
                t1W_ref, t1b_ref, t2W_ref, t2b_ref,
                z_ref, lv_ref, y_ref, xr_ref, zs_ref):
    i = pl.program_id(0)
    dis = _dis(deg_ref)
    h = _gcn_post(hom_ref, acc_ref, hws_ref, dis, gb_ref, g_ref, bb_ref,
                  hw3_ref, hb_ref)
    mub = _hcvec(hom_ref, muH_ref, mub_ref)
    lvb = _hcvec(hom_ref, lvH_ref, lvb_ref)
    z = jnp.dot(h, muW_ref[...], preferred_element_type=jnp.float32) + mub
    z_ref[...] = z
    lv_ref[...] = jnp.dot(h, lvW_ref[...],
                          preferred_element_type=jnp.float32) + lvb
    t = jnp.maximum(jnp.dot(z, l1W_ref[...],
                            preferred_element_type=jnp.float32)
                    + l1b_ref[...], 0.0)
    y_ref[...] = jnp.dot(t, l2W_ref[...],
                         preferred_element_type=jnp.float32) + l2b_ref[...]
    zp = (jnp.dot(z, pW_ref[...], preferred_element_type=jnp.float32)
          + pb_ref[...]) * (pg_ref[...] * BNK) + pbb_ref[...]
    xr = jnp.maximum(jnp.dot(zp, t1W_ref[...],
                             preferred_element_type=jnp.float32)
                     + t1b_ref[...], 0.0)
    xr_ref[...] = jnp.dot(xr, t2W_ref[...],
                          preferred_element_type=jnp.float32) + t2b_ref[...]
    part = jnp.sum(z, axis=0, keepdims=True)

    @pl.when(i == 0)
    def _():
        zs_ref[...] = part

    @pl.when(i != 0)
    def _():
        zs_ref[...] = zs_ref[...] + part


def _final(hom, acc, hws, deg16, gb, g, bb, hw3, hb,
           muW, muH, mub, lvW, lvH, lvb, l1W, l1b, l2W, l2b,
           pW, pb, pg, pbb, t1W, t1b, t2W, t2b):
    T = 1000
    NCLS = 7
    TLAT = 128
    full = lambda r, c: pl.BlockSpec((r, c), lambda i: (0, 0))
    row = lambda c: pl.BlockSpec((T, c), lambda i: (i, 0))
    return pl.pallas_call(
        _final_body,
        grid=(N // T,),
        in_specs=[
            pl.BlockSpec(memory_space=pltpu.SMEM),
            row(FEAT), row(FEAT), pl.BlockSpec((T, 16), lambda i: (i, 0)),
            full(1, FEAT), full(1, FEAT), full(1, FEAT),
            full(3, FEAT), full(1, FEAT),
            full(FEAT, LAT), full(3, LAT), full(1, LAT),
            full(FEAT, LAT), full(3, LAT), full(1, LAT),
            full(LAT, LAT), full(1, LAT), full(LAT, NCLS), full(1, NCLS),
            full(LAT, TLAT), full(1, TLAT), full(1, TLAT), full(1, TLAT),
            full(TLAT, FEAT), full(1, FEAT), full(FEAT, FEAT), full(1, FEAT),
        ],
        out_specs=[
            row(LAT), row(LAT), row(NCLS), row(FEAT),
            pl.BlockSpec((1, LAT), lambda i: (0, 0)),
        ],
        out_shape=[
            jax.ShapeDtypeStruct((N, LAT), jnp.float32),
            jax.ShapeDtypeStruct((N, LAT), jnp.float32),
            jax.ShapeDtypeStruct((N, NCLS), jnp.float32),
            jax.ShapeDtypeStruct((N, FEAT), jnp.float32),
            jax.ShapeDtypeStruct((1, LAT), jnp.float32),
        ],
    )(hom, acc, hws, deg16, gb, g, bb, hw3, hb,
      muW, muH, mub, lvW, lvH, lvb, l1W, l1b, l2W, l2b,
      pW, pb, pg, pbb, t1W, t1b, t2W, t2b)


def _adj_body(zr_ref, zc_ref, o_ref):
    o_ref[...] = jax.nn.sigmoid(
        lax.dot_general(zr_ref[...], zc_ref[...], (((1,), (1,)), ((), ())),
                        preferred_element_type=jnp.float32))


def _adj(z):
    TR, TCOL = 1000, 2048
    return pl.pallas_call(
        _adj_body,
        grid=(N // TR, pl.cdiv(N, TCOL)),
        in_specs=[
            pl.BlockSpec((TR, LAT), lambda i, j: (i, 0)),
            pl.BlockSpec((TCOL, LAT), lambda i, j: (j, 0)),
        ],
        out_specs=pl.BlockSpec((TR, TCOL), lambda i, j: (i, j)),
        out_shape=jax.ShapeDtypeStruct((N, N), jnp.float32),
    )(z, z)


def _heads_body(zs_ref, lh1, lh1b, lh2, lh2b, sh1, sh1b, sh2, sh2b,
                fh1, fh1b, fh2, fh2b, o_ref):
    zg = zs_ref[...] * (1.0 / N)

    def head(w1, b1, w2, b2):
        t = jnp.maximum(jnp.dot(zg, w1[...],
                                preferred_element_type=jnp.float32)
                        + b1[...], 0.0)
        return jnp.dot(t, w2[...], preferred_element_type=jnp.float32) + b2[...]

    lh = jax.nn.sigmoid(head(lh1, lh1b, lh2, lh2b))
    sh = jax.nn.sigmoid(head(sh1, sh1b, sh2, sh2b))
    fh = jnp.tanh(head(fh1, fh1b, fh2, fh2b))
    o_ref[...] = jnp.concatenate([lh, sh, fh], axis=1)


def _heads(zsum, lh1, lh1b, lh2, lh2b, sh1, sh1b, sh2, sh2b,
           fh1, fh1b, fh2, fh2b):
    full = lambda r, c: pl.BlockSpec((r, c), lambda: (0, 0))
    return pl.pallas_call(
        _heads_body,
        in_specs=[full(1, LAT),
                  full(LAT, LAT), full(1, LAT), full(LAT, 1), full(1, 1),
                  full(LAT, LAT), full(1, LAT), full(LAT, 1), full(1, 1),
                  full(LAT, LAT), full(1, LAT), full(LAT, 1), full(1, 1)],
        out_specs=full(1, 3),
        out_shape=jax.ShapeDtypeStruct((1, 3), jnp.float32),
    )(zsum, lh1, lh1b, lh2, lh2b, sh1, sh1b, sh2, sh2b, fh1, fh1b, fh2, fh2b)


def kernel(x, edge_index, homophily_cond, batch, params):
    p = params
    f32 = jnp.float32
    src = edge_index[0]
    dst = edge_index[1]
    src_p = jnp.concatenate([src, jnp.zeros((E_PAD - E_MSG,), jnp.int32)])
    dst_p = jnp.concatenate([dst, jnp.full((E_PAD - E_MSG,), PAD_DST,
                                           jnp.int32)])
    zacc = jnp.zeros((ACC_PT, FEAT), f32)
    hom = homophily_cond
    r = lambda v: v.reshape(1, -1)

    srcs, dstl, cnts, deg16 = _part_call(src_p, dst_p)
    hw0s = _mmscale(x, p['gcn0_W'], deg16)
    acc0 = _scatter_call(hw0s, srcs, dstl, cnts, zacc)
    hw1s = _layer(hom, acc0, hw0s, deg16,
                  r(p['gcn0_b']), r(p['bn0_g']), r(p['bn0_b']),
                  p['hom0_W'], r(p['hom0_b']), p['gcn1_W'])
    acc1 = _scatter_call(hw1s, srcs, dstl, cnts, zacc)
    z, lv, y, xr, zsum = _final(
        hom, acc1, hw1s, deg16,
        r(p['gcn1_b']), r(p['bn1_g']), r(p['bn1_b']),
        p['hom1_W'], r(p['hom1_b']),
        p['mu_W'][:FEAT], p['mu_W'][FEAT:], r(p['mu_b']),
        p['lv_W'][:FEAT], p['lv_W'][FEAT:], r(p['lv_b']),
        p['lab1_W'], r(p['lab1_b']), p['lab2_W'], r(p['lab2_b']),
        p['proj_W'], r(p['proj_b']), r(p['projbn_g']), r(p['projbn_b']),
        p['t1_W'], r(p['t1_b']), p['t2_W'], r(p['t2_b']))
    adj = _adj(z)
    homp = _heads(zsum,
                  p['lh1_W'], r(p['lh1_b']), p['lh2_W'], r(p['lh2_b']),
                  p['sh1_W'], r(p['sh1_b']), p['sh2_W'], r(p['sh2_b']),
                  p['fh1_W'], r(p['fh1_b']), p['fh2_W'], r(p['fh2_b']))
    return (adj, xr, y, homp, z, lv)
